# baseline (speedup 1.0000x reference)
import jax
import jax.numpy as jnp
from jax import lax
from jax.experimental import pallas as pl
from jax.experimental.pallas import tpu as pltpu
from jax.experimental.pallas import tpu_sc as plsc

NC = 2
NS = 16
NW = NC * NS
L = 16

NPLANE = 49
PSIZE = 256 * 2048
NTILE = PSIZE // 1024
T0 = 128
E_SC = T0 * 1024
PER_W = E_SC // NW
CH = 1024
NCHUNK = PER_W // CH
GROUPS = CH // L
TROW = (NTILE - T0) // 16 * 8


def _sort4(a, b, c, d):
    a, b = jnp.maximum(a, b), jnp.minimum(a, b)
    c, d = jnp.maximum(c, d), jnp.minimum(c, d)
    a, c = jnp.maximum(a, c), jnp.minimum(a, c)
    b, d = jnp.maximum(b, d), jnp.minimum(b, d)
    b, c = jnp.maximum(b, c), jnp.minimum(b, c)
    return a, b, c, d


def _merge44(A, B):
    a1, a2, a3, a4 = A
    b1, b2, b3, b4 = B
    c1 = jnp.maximum(a1, b1)
    q = jnp.minimum(a1, b1)
    r = jnp.maximum(a3, b3)
    c2 = jnp.maximum(q, r)
    c3 = jnp.minimum(q, r)
    d1 = jnp.maximum(a2, b2)
    q2 = jnp.minimum(a2, b2)
    r2 = jnp.maximum(a4, b4)
    d2 = jnp.maximum(q2, r2)
    return c1, jnp.maximum(d1, c2), jnp.minimum(d1, c2), jnp.maximum(d2, c3)


def _insert4(m, v):
    m1, m2, m3, m4 = m
    t = jnp.minimum(m1, v)
    m1 = jnp.maximum(m1, v)
    u = jnp.minimum(m2, t)
    m2 = jnp.maximum(m2, t)
    w = jnp.minimum(m3, u)
    m3 = jnp.maximum(m3, u)
    m4 = jnp.maximum(m4, w)
    return m1, m2, m3, m4


def _top4mean(load):
    def t4(p0):
        return _sort4(load(p0), load(p0 + 1), load(p0 + 2), load(p0 + 3))

    def t16(p0):
        return _merge44(_merge44(t4(p0), t4(p0 + 4)),
                        _merge44(t4(p0 + 8), t4(p0 + 12)))

    t = _merge44(_merge44(t16(0), t16(16)), t16(32))
    t = _insert4(t, load(48))
    return (t[0] + t[1] + t[2] + t[3]) * 0.25


def _sc_body(x_hbm, out_hbm, in0, in1, oacc, si0, si1, so):
    wid = lax.axis_index("s") * NC + lax.axis_index("c")
    wbase = wid * PER_W

    bufs = (in0, in1)
    sems = (si0, si1)

    def issue(c, b):
        base = wbase + c * CH
        for p in range(NPLANE):
            pltpu.async_copy(
                x_hbm.at[pl.ds(p * PSIZE + base, CH)],
                bufs[b].at[pl.ds(p * CH, CH)],
                sems[b],
            )

    def drain(b):
        pltpu.make_async_copy(
            x_hbm.at[pl.ds(0, NPLANE * CH)], bufs[b], sems[b]
        ).wait()

    def compute(c, b):
        buf = bufs[b]

        def group2(i, carry):
            g = 2 * i

            def res(gg):
                return _top4mean(lambda p: buf[pl.ds(p * CH + gg * L, L)])

            o = ((g >> 3) * NCHUNK << 7) + (c << 7) + ((g & 7) << 4)
            oacc[pl.ds(o, L)] = res(g)
            g1 = g + 1
            o1 = ((g1 >> 3) * NCHUNK << 7) + (c << 7) + ((g1 & 7) << 4)
            oacc[pl.ds(o1, L)] = res(g1)
            return carry

        lax.fori_loop(0, GROUPS // 2, group2, 0)

    issue(0, 0)

    def pair(i, carry):
        c0 = 2 * i
        issue(c0 + 1, 1)
        drain(0)
        compute(c0, 0)

        @pl.when(i < NCHUNK // 2 - 1)
        def _():
            issue(c0 + 2, 0)

        drain(1)
        compute(c0 + 1, 1)
        return carry

    lax.fori_loop(0, NCHUNK // 2, pair, 0)

    wcol = NCHUNK << 7
    obase = (wid // (16 // NCHUNK)) * 8 * 2048 + (wid * NCHUNK % 16) * 128
    for r in range(8):
        pltpu.async_copy(
            oacc.at[pl.ds(r * wcol, wcol)],
            out_hbm.at[pl.ds(obase + r * 2048, wcol)],
            so,
        )
    pltpu.make_async_copy(
        x_hbm.at[pl.ds(0, 8 * wcol)], oacc, so
    ).wait()


@jax.jit
def _apool_sc(xf):
    mesh = plsc.VectorSubcoreMesh(
        core_axis_name="c", subcore_axis_name="s", num_cores=NC, num_subcores=NS
    )
    return pl.kernel(
        _sc_body,
        out_type=jax.ShapeDtypeStruct((E_SC,), jnp.float32),
        mesh=mesh,
        compiler_params=pltpu.CompilerParams(needs_layout_passes=False),
        scratch_types=[
            pltpu.VMEM((NPLANE * CH,), jnp.float32),
            pltpu.VMEM((NPLANE * CH,), jnp.float32),
            pltpu.VMEM((PER_W,), jnp.float32),
            pltpu.SemaphoreType.DMA,
            pltpu.SemaphoreType.DMA,
            pltpu.SemaphoreType.DMA,
        ],
    )(xf)


def _tc_block(x_ref, o_ref):
    o_ref[...] = _top4mean(lambda p: x_ref[p])


@jax.jit
def _apool_tc(x3):
    return pl.pallas_call(
        _tc_block,
        grid=(NTILE - T0,),
        in_specs=[
            pl.BlockSpec((NPLANE, 8, 128), lambda t: (0, t + T0, 0)),
        ],
        out_specs=pl.BlockSpec((8, 128), lambda t: ((t + T0) // 16 - T0 // 16,
                                                    (t + T0) % 16)),
        out_shape=jax.ShapeDtypeStruct((TROW, 2048), jnp.float32),
    )(x3)


def kernel(input, dim):
    xt = jnp.transpose(input, (2, 3, 0, 1))
    x5 = xt.reshape(49, 32, 8, 16, 128)
    x5 = jnp.transpose(x5, (0, 1, 3, 2, 4))
    xf = x5.reshape(-1)
    osc = _apool_sc(xf)
    otc = _apool_tc(xf.reshape(NPLANE, PSIZE // 128, 128))
    out = jnp.concatenate([osc, otc.reshape(-1)])
    return out.reshape(256, 2048, 1, 1)


# speedup vs baseline: 3.8585x; 3.8585x over previous
import jax
import jax.numpy as jnp
from jax import lax
from jax.experimental import pallas as pl
from jax.experimental.pallas import tpu as pltpu
from jax.experimental.pallas import tpu_sc as plsc

NC = 2
NS = 16
NW = NC * NS
L = 16

NPLANE = 49
PSIZE = 256 * 2048
NTILE = PSIZE // 1024
T0 = 128
E_SC = T0 * 1024
PER_W = E_SC // NW
CH = 1024
NCHUNK = PER_W // CH
GROUPS = CH // L
TROW = (NTILE - T0) // 16 * 8


def _sort4(a, b, c, d):
    a, b = jnp.maximum(a, b), jnp.minimum(a, b)
    c, d = jnp.maximum(c, d), jnp.minimum(c, d)
    a, c = jnp.maximum(a, c), jnp.minimum(a, c)
    b, d = jnp.maximum(b, d), jnp.minimum(b, d)
    b, c = jnp.maximum(b, c), jnp.minimum(b, c)
    return a, b, c, d


def _merge44(A, B):
    a1, a2, a3, a4 = A
    b1, b2, b3, b4 = B
    c1 = jnp.maximum(a1, b1)
    q = jnp.minimum(a1, b1)
    r = jnp.maximum(a3, b3)
    c2 = jnp.maximum(q, r)
    c3 = jnp.minimum(q, r)
    d1 = jnp.maximum(a2, b2)
    q2 = jnp.minimum(a2, b2)
    r2 = jnp.maximum(a4, b4)
    d2 = jnp.maximum(q2, r2)
    return c1, jnp.maximum(d1, c2), jnp.minimum(d1, c2), jnp.maximum(d2, c3)


def _insert4(m, v):
    m1, m2, m3, m4 = m
    t = jnp.minimum(m1, v)
    m1 = jnp.maximum(m1, v)
    u = jnp.minimum(m2, t)
    m2 = jnp.maximum(m2, t)
    w = jnp.minimum(m3, u)
    m3 = jnp.maximum(m3, u)
    m4 = jnp.maximum(m4, w)
    return m1, m2, m3, m4


def _top4mean(load):
    def t4(p0):
        return _sort4(load(p0), load(p0 + 1), load(p0 + 2), load(p0 + 3))

    def t16(p0):
        return _merge44(_merge44(t4(p0), t4(p0 + 4)),
                        _merge44(t4(p0 + 8), t4(p0 + 12)))

    t = _merge44(_merge44(t16(0), t16(16)), t16(32))
    t = _insert4(t, load(48))
    return (t[0] + t[1] + t[2] + t[3]) * 0.25


def _sc_body(x_hbm, out_hbm, in0, in1, oacc, si0, si1, so):
    wid = lax.axis_index("s") * NC + lax.axis_index("c")
    wbase = wid * PER_W

    bufs = (in0, in1)
    sems = (si0, si1)

    def issue(c, b):
        base = wbase + c * CH
        for p in range(NPLANE):
            pltpu.async_copy(
                x_hbm.at[pl.ds(p * PSIZE + base, CH)],
                bufs[b].at[pl.ds(p * CH, CH)],
                sems[b],
            )

    def drain(b):
        pltpu.make_async_copy(
            x_hbm.at[pl.ds(0, NPLANE * CH)], bufs[b], sems[b]
        ).wait()

    def compute(c, b):
        buf = bufs[b]

        def group2(i, carry):
            g = 2 * i

            def res(gg):
                return _top4mean(lambda p: buf[pl.ds(p * CH + gg * L, L)])

            o = ((g >> 3) * NCHUNK << 7) + (c << 7) + ((g & 7) << 4)
            oacc[pl.ds(o, L)] = res(g)
            g1 = g + 1
            o1 = ((g1 >> 3) * NCHUNK << 7) + (c << 7) + ((g1 & 7) << 4)
            oacc[pl.ds(o1, L)] = res(g1)
            return carry

        lax.fori_loop(0, GROUPS // 2, group2, 0)

    issue(0, 0)

    def pair(i, carry):
        c0 = 2 * i
        issue(c0 + 1, 1)
        drain(0)
        compute(c0, 0)

        @pl.when(i < NCHUNK // 2 - 1)
        def _():
            issue(c0 + 2, 0)

        drain(1)
        compute(c0 + 1, 1)
        return carry

    lax.fori_loop(0, NCHUNK // 2, pair, 0)

    wcol = NCHUNK << 7
    obase = (wid // (16 // NCHUNK)) * 8 * 2048 + (wid * NCHUNK % 16) * 128
    for r in range(8):
        pltpu.async_copy(
            oacc.at[pl.ds(r * wcol, wcol)],
            out_hbm.at[pl.ds(obase + r * 2048, wcol)],
            so,
        )
    pltpu.make_async_copy(
        x_hbm.at[pl.ds(0, 8 * wcol)], oacc, so
    ).wait()


@jax.jit
def _apool_sc(xf):
    mesh = plsc.VectorSubcoreMesh(
        core_axis_name="c", subcore_axis_name="s", num_cores=NC, num_subcores=NS
    )
    return pl.kernel(
        _sc_body,
        out_type=jax.ShapeDtypeStruct((E_SC,), jnp.float32),
        mesh=mesh,
        compiler_params=pltpu.CompilerParams(needs_layout_passes=False),
        scratch_types=[
            pltpu.VMEM((NPLANE * CH,), jnp.float32),
            pltpu.VMEM((NPLANE * CH,), jnp.float32),
            pltpu.VMEM((PER_W,), jnp.float32),
            pltpu.SemaphoreType.DMA,
            pltpu.SemaphoreType.DMA,
            pltpu.SemaphoreType.DMA,
        ],
    )(xf)


def _tc_block(x_ref, o_ref):
    res = _top4mean(lambda p: x_ref[p])
    for ch in range(16):
        o_ref[:, ch * 128:(ch + 1) * 128] = res[ch * 8:(ch + 1) * 8, :]


@jax.jit
def _apool_tc(x3):
    return pl.pallas_call(
        _tc_block,
        grid=((NTILE - T0) // 16,),
        in_specs=[
            pl.BlockSpec((NPLANE, 128, 128), lambda t: (0, t + T0 // 16, 0)),
        ],
        out_specs=pl.BlockSpec((8, 2048), lambda t: (t, 0)),
        out_shape=jax.ShapeDtypeStruct((TROW, 2048), jnp.float32),
    )(x3)


def kernel(input, dim):
    xt = jnp.transpose(input, (2, 3, 0, 1))
    x5 = xt.reshape(49, 32, 8, 16, 128)
    x5 = jnp.transpose(x5, (0, 1, 3, 2, 4))
    xf = x5.reshape(-1)
    osc = _apool_sc(xf)
    otc = _apool_tc(xf.reshape(NPLANE, PSIZE // 128, 128))
    out = jnp.concatenate([osc, otc.reshape(-1)])
    return out.reshape(256, 2048, 1, 1)


# speedup vs baseline: 4.1342x; 1.0714x over previous
import jax
import jax.numpy as jnp
from jax import lax
from jax.experimental import pallas as pl
from jax.experimental.pallas import tpu as pltpu
from jax.experimental.pallas import tpu_sc as plsc

NC = 2
NS = 16
NW = NC * NS
L = 16

NPLANE = 49
PSIZE = 256 * 2048
NTILE = PSIZE // 1024
T0 = 128
E_SC = T0 * 1024
PER_W = E_SC // NW
CH = 1024
NCHUNK = PER_W // CH
GROUPS = CH // L
TROW = (NTILE - T0) // 16 * 8


def _sort4(a, b, c, d):
    a, b = jnp.maximum(a, b), jnp.minimum(a, b)
    c, d = jnp.maximum(c, d), jnp.minimum(c, d)
    a, c = jnp.maximum(a, c), jnp.minimum(a, c)
    b, d = jnp.maximum(b, d), jnp.minimum(b, d)
    b, c = jnp.maximum(b, c), jnp.minimum(b, c)
    return a, b, c, d


def _merge44(A, B):
    a1, a2, a3, a4 = A
    b1, b2, b3, b4 = B
    c1 = jnp.maximum(a1, b1)
    q = jnp.minimum(a1, b1)
    r = jnp.maximum(a3, b3)
    c2 = jnp.maximum(q, r)
    c3 = jnp.minimum(q, r)
    d1 = jnp.maximum(a2, b2)
    q2 = jnp.minimum(a2, b2)
    r2 = jnp.maximum(a4, b4)
    d2 = jnp.maximum(q2, r2)
    return c1, jnp.maximum(d1, c2), jnp.minimum(d1, c2), jnp.maximum(d2, c3)


def _insert4(m, v):
    m1, m2, m3, m4 = m
    t = jnp.minimum(m1, v)
    m1 = jnp.maximum(m1, v)
    u = jnp.minimum(m2, t)
    m2 = jnp.maximum(m2, t)
    w = jnp.minimum(m3, u)
    m3 = jnp.maximum(m3, u)
    m4 = jnp.maximum(m4, w)
    return m1, m2, m3, m4


def _top4mean(load):
    def t4(p0):
        return _sort4(load(p0), load(p0 + 1), load(p0 + 2), load(p0 + 3))

    def t16(p0):
        return _merge44(_merge44(t4(p0), t4(p0 + 4)),
                        _merge44(t4(p0 + 8), t4(p0 + 12)))

    t = _merge44(_merge44(t16(0), t16(16)), t16(32))
    t = _insert4(t, load(48))
    return (t[0] + t[1] + t[2] + t[3]) * 0.25


def _sc_body(x_hbm, out_hbm, in0, in1, oacc, si0, si1, so):
    wid = lax.axis_index("s") * NC + lax.axis_index("c")
    wbase = wid * PER_W

    bufs = (in0, in1)
    sems = (si0, si1)

    def issue(c, b):
        base = wbase + c * CH
        for p in range(NPLANE):
            pltpu.async_copy(
                x_hbm.at[pl.ds(p * PSIZE + base, CH)],
                bufs[b].at[pl.ds(p * CH, CH)],
                sems[b],
            )

    def drain(b):
        pltpu.make_async_copy(
            x_hbm.at[pl.ds(0, NPLANE * CH)], bufs[b], sems[b]
        ).wait()

    def compute(c, b):
        buf = bufs[b]

        def group2(i, carry):
            g = 2 * i

            def res(gg):
                return _top4mean(lambda p: buf[pl.ds(p * CH + gg * L, L)])

            o = ((g >> 3) * NCHUNK << 7) + (c << 7) + ((g & 7) << 4)
            oacc[pl.ds(o, L)] = res(g)
            g1 = g + 1
            o1 = ((g1 >> 3) * NCHUNK << 7) + (c << 7) + ((g1 & 7) << 4)
            oacc[pl.ds(o1, L)] = res(g1)
            return carry

        lax.fori_loop(0, GROUPS // 2, group2, 0)

    issue(0, 0)

    def pair(i, carry):
        c0 = 2 * i
        issue(c0 + 1, 1)
        drain(0)
        compute(c0, 0)

        @pl.when(i < NCHUNK // 2 - 1)
        def _():
            issue(c0 + 2, 0)

        drain(1)
        compute(c0 + 1, 1)
        return carry

    lax.fori_loop(0, NCHUNK // 2, pair, 0)

    wcol = NCHUNK << 7
    obase = (wid // (16 // NCHUNK)) * 8 * 2048 + (wid * NCHUNK % 16) * 128
    for r in range(8):
        pltpu.async_copy(
            oacc.at[pl.ds(r * wcol, wcol)],
            out_hbm.at[pl.ds(obase + r * 2048, wcol)],
            so,
        )
    pltpu.make_async_copy(
        x_hbm.at[pl.ds(0, 8 * wcol)], oacc, so
    ).wait()


@jax.jit
def _apool_sc(xf):
    mesh = plsc.VectorSubcoreMesh(
        core_axis_name="c", subcore_axis_name="s", num_cores=NC, num_subcores=NS
    )
    return pl.kernel(
        _sc_body,
        out_type=jax.ShapeDtypeStruct((E_SC,), jnp.float32),
        mesh=mesh,
        compiler_params=pltpu.CompilerParams(needs_layout_passes=False),
        scratch_types=[
            pltpu.VMEM((NPLANE * CH,), jnp.float32),
            pltpu.VMEM((NPLANE * CH,), jnp.float32),
            pltpu.VMEM((PER_W,), jnp.float32),
            pltpu.SemaphoreType.DMA,
            pltpu.SemaphoreType.DMA,
            pltpu.SemaphoreType.DMA,
        ],
    )(xf)


def _tc_block(x_ref, o_ref):
    res = _top4mean(lambda p: x_ref[p])
    for j in range(32):
        o_ref[(j // 16) * 8:(j // 16) * 8 + 8, (j % 16) * 128:(j % 16) * 128 + 128] = (
            res[j * 8:(j + 1) * 8, :]
        )


@jax.jit
def _apool_tc(x3):
    return pl.pallas_call(
        _tc_block,
        grid=((NTILE - T0) // 32,),
        in_specs=[
            pl.BlockSpec((NPLANE, 256, 128), lambda t: (0, t + T0 // 32, 0)),
        ],
        out_specs=pl.BlockSpec((16, 2048), lambda t: (t, 0)),
        out_shape=jax.ShapeDtypeStruct((TROW, 2048), jnp.float32),
    )(x3)


def kernel(input, dim):
    xt = jnp.transpose(input, (2, 3, 0, 1))
    x5 = xt.reshape(49, 32, 8, 16, 128)
    x5 = jnp.transpose(x5, (0, 1, 3, 2, 4))
    xf = x5.reshape(-1)
    osc = _apool_sc(xf)
    otc = _apool_tc(xf.reshape(NPLANE, PSIZE // 128, 128))
    out = jnp.concatenate([osc, otc.reshape(-1)])
    return out.reshape(256, 2048, 1, 1)


# speedup vs baseline: 4.1884x; 1.0131x over previous
import jax
import jax.numpy as jnp
from jax import lax
from jax.experimental import pallas as pl
from jax.experimental.pallas import tpu as pltpu
from jax.experimental.pallas import tpu_sc as plsc

NC = 2
NS = 16
NW = NC * NS
L = 16

NPLANE = 49
PSIZE = 256 * 2048
NTILE = PSIZE // 1024
T0 = 128
E_SC = T0 * 1024
PER_W = E_SC // NW
CH = 1024
NCHUNK = PER_W // CH
GROUPS = CH // L
TROW = (NTILE - T0) // 16 * 8


def _sort4(a, b, c, d):
    a, b = jnp.maximum(a, b), jnp.minimum(a, b)
    c, d = jnp.maximum(c, d), jnp.minimum(c, d)
    a, c = jnp.maximum(a, c), jnp.minimum(a, c)
    b, d = jnp.maximum(b, d), jnp.minimum(b, d)
    b, c = jnp.maximum(b, c), jnp.minimum(b, c)
    return a, b, c, d


def _merge44(A, B):
    a1, a2, a3, a4 = A
    b1, b2, b3, b4 = B
    c1 = jnp.maximum(a1, b1)
    q = jnp.minimum(a1, b1)
    r = jnp.maximum(a3, b3)
    c2 = jnp.maximum(q, r)
    c3 = jnp.minimum(q, r)
    d1 = jnp.maximum(a2, b2)
    q2 = jnp.minimum(a2, b2)
    r2 = jnp.maximum(a4, b4)
    d2 = jnp.maximum(q2, r2)
    return c1, jnp.maximum(d1, c2), jnp.minimum(d1, c2), jnp.maximum(d2, c3)


def _insert4(m, v):
    m1, m2, m3, m4 = m
    t = jnp.minimum(m1, v)
    m1 = jnp.maximum(m1, v)
    u = jnp.minimum(m2, t)
    m2 = jnp.maximum(m2, t)
    w = jnp.minimum(m3, u)
    m3 = jnp.maximum(m3, u)
    m4 = jnp.maximum(m4, w)
    return m1, m2, m3, m4


def _top4mean(load):
    def t4(p0):
        return _sort4(load(p0), load(p0 + 1), load(p0 + 2), load(p0 + 3))

    def t16(p0):
        return _merge44(_merge44(t4(p0), t4(p0 + 4)),
                        _merge44(t4(p0 + 8), t4(p0 + 12)))

    t = _merge44(_merge44(t16(0), t16(16)), t16(32))
    t = _insert4(t, load(48))
    return (t[0] + t[1] + t[2] + t[3]) * 0.25


def _sc_body(x_hbm, out_hbm, in0, in1, oacc, si0, si1, so):
    wid = lax.axis_index("s") * NC + lax.axis_index("c")
    wbase = wid * PER_W

    bufs = (in0, in1)
    sems = (si0, si1)

    def issue(c, b):
        base = wbase + c * CH
        for p in range(NPLANE):
            pltpu.async_copy(
                x_hbm.at[pl.ds(p * PSIZE + base, CH)],
                bufs[b].at[pl.ds(p * CH, CH)],
                sems[b],
            )

    def drain(b):
        pltpu.make_async_copy(
            x_hbm.at[pl.ds(0, NPLANE * CH)], bufs[b], sems[b]
        ).wait()

    def compute(c, b):
        buf = bufs[b]

        def group2(i, carry):
            g = 2 * i

            def res(gg):
                return _top4mean(lambda p: buf[pl.ds(p * CH + gg * L, L)])

            o = ((g >> 3) * NCHUNK << 7) + (c << 7) + ((g & 7) << 4)
            oacc[pl.ds(o, L)] = res(g)
            g1 = g + 1
            o1 = ((g1 >> 3) * NCHUNK << 7) + (c << 7) + ((g1 & 7) << 4)
            oacc[pl.ds(o1, L)] = res(g1)
            return carry

        lax.fori_loop(0, GROUPS // 2, group2, 0)

    issue(0, 0)

    def pair(i, carry):
        c0 = 2 * i
        issue(c0 + 1, 1)
        drain(0)
        compute(c0, 0)

        @pl.when(i < NCHUNK // 2 - 1)
        def _():
            issue(c0 + 2, 0)

        drain(1)
        compute(c0 + 1, 1)
        return carry

    lax.fori_loop(0, NCHUNK // 2, pair, 0)

    wcol = NCHUNK << 7
    obase = (wid // (16 // NCHUNK)) * 8 * 2048 + (wid * NCHUNK % 16) * 128
    for r in range(8):
        pltpu.async_copy(
            oacc.at[pl.ds(r * wcol, wcol)],
            out_hbm.at[pl.ds(obase + r * 2048, wcol)],
            so,
        )
    pltpu.make_async_copy(
        x_hbm.at[pl.ds(0, 8 * wcol)], oacc, so
    ).wait()


@jax.jit
def _apool_sc(xf):
    mesh = plsc.VectorSubcoreMesh(
        core_axis_name="c", subcore_axis_name="s", num_cores=NC, num_subcores=NS
    )
    return pl.kernel(
        _sc_body,
        out_type=jax.ShapeDtypeStruct((E_SC,), jnp.float32),
        mesh=mesh,
        compiler_params=pltpu.CompilerParams(
            needs_layout_passes=False, skip_device_barrier=True
        ),
        scratch_types=[
            pltpu.VMEM((NPLANE * CH,), jnp.float32),
            pltpu.VMEM((NPLANE * CH,), jnp.float32),
            pltpu.VMEM((PER_W,), jnp.float32),
            pltpu.SemaphoreType.DMA,
            pltpu.SemaphoreType.DMA,
            pltpu.SemaphoreType.DMA,
        ],
    )(xf)


TCR = 512


def _tc_block(x_ref, o_ref):
    res = _top4mean(lambda p: x_ref[p])
    for j in range(TCR // 8):
        o_ref[(j // 16) * 8:(j // 16) * 8 + 8,
              (j % 16) * 128:(j % 16) * 128 + 128] = res[j * 8:(j + 1) * 8, :]


@jax.jit
def _apool_tc(x3):
    return pl.pallas_call(
        _tc_block,
        grid=((NTILE - T0) * 8 // TCR,),
        in_specs=[
            pl.BlockSpec((NPLANE, TCR, 128), lambda t: (0, t + T0 * 8 // TCR, 0)),
        ],
        out_specs=pl.BlockSpec((TCR // 16, 2048), lambda t: (t, 0)),
        out_shape=jax.ShapeDtypeStruct((TROW, 2048), jnp.float32),
    )(x3)


def kernel(input, dim):
    xt = jnp.transpose(input, (2, 3, 0, 1))
    x5 = xt.reshape(49, 32, 8, 16, 128)
    x5 = jnp.transpose(x5, (0, 1, 3, 2, 4))
    xf = x5.reshape(-1)
    osc = _apool_sc(xf)
    otc = _apool_tc(xf.reshape(NPLANE, PSIZE // 128, 128))
    out = jnp.concatenate([osc, otc.reshape(-1)])
    return out.reshape(256, 2048, 1, 1)
